# baseline (speedup 1.0000x reference)
import functools

import jax
import jax.numpy as jnp
from jax import lax
from jax.experimental import pallas as pl
from jax.experimental.pallas import tpu as pltpu
from jax.experimental.pallas import tpu_sc as plsc

_B, _F, _T = 64, 128, 4096
_HS = 64
_NC, _NS = 2, 16
_NW = _NC * _NS
_BPW = _B // _NW
_RC = 8
_NCH = _F // _RC
_ML = 16
_WR, _WC = 72, 256


def _window(f, t):
    fa = pl.multiple_of(jnp.minimum(f & ~7, _F - _WR), 8)
    ta = pl.multiple_of(jnp.minimum(t & ~127, _T - _WC), 128)
    return fa, ta


def _patch_window(wbuf, fa, ta, f, t):
    zv = jnp.zeros((16,), jnp.float32)
    li = lax.broadcasted_iota(jnp.int32, (16,), 0)
    lo = jnp.maximum(f - fa, 0)
    hi = jnp.minimum(f + _HS - fa, _WR)
    cb = t - ta

    def row_body(r, carry):
        rows = jnp.full((16,), r, jnp.int32)
        for j in range(_HS // 16):
            cols = cb + j * 16 + li
            m = (cols >= 0) & (cols < _WC)
            colsc = jnp.clip(cols, 0, _WC - 1)
            plsc.store_scatter(wbuf, [rows, colsc], zv, mask=m)
        return carry

    lax.fori_loop(lo, hi, row_body, 0)


def _sc_body(x_hbm, meta_hbm, out_hbm, sbuf, wb0, meta_v,
             isem0, isem1, isem2, osem0, osem1, osem2,
             wl0, ws0):
    wid = lax.axis_index("s") * _NC + lax.axis_index("c")
    sid = lax.axis_index("s")
    pltpu.sync_copy(meta_hbm, meta_v)
    bufs = (sbuf.at[sid, 0], sbuf.at[sid, 1], sbuf.at[sid, 2])
    isems = (isem0, isem1, isem2)
    osems = (osem0, osem1, osem2)
    wbufs = (wb0,)
    wlsems = (wl0,)
    wssems = (ws0,)

    holes = []
    wins = []
    for bi in range(_BPW):
        b = wid * _BPW + bi
        mv = meta_v[b]
        bh = []
        for h in range(2):
            f, t = mv[h], mv[2 + h]
            bh.append((f, t))
            wins.append(_window(f, t))
        holes.append(bh)

    def wload(w):
        b = wid * _BPW + w // 2
        fa, ta = wins[w]
        pltpu.make_async_copy(
            x_hbm.at[b, pl.ds(fa, _WR), pl.ds(ta, _WC)],
            wbufs[0], wlsems[0]).start()

    def wload_wait(w):
        b = wid * _BPW + w // 2
        fa, ta = wins[w]
        pltpu.make_async_copy(
            x_hbm.at[b, pl.ds(fa, _WR), pl.ds(ta, _WC)],
            wbufs[0], wlsems[0]).wait()

    def wstore(w):
        b = wid * _BPW + w // 2
        fa, ta = wins[w]
        pltpu.make_async_copy(
            wbufs[0], out_hbm.at[b, pl.ds(fa, _WR), pl.ds(ta, _WC)],
            wssems[0]).start()

    def wstore_wait(w):
        b = wid * _BPW + w // 2
        fa, ta = wins[w]
        pltpu.make_async_copy(
            wbufs[0], out_hbm.at[b, pl.ds(fa, _WR), pl.ds(ta, _WC)],
            wssems[0]).wait()

    n_items = _BPW * _NCH

    def item(i):
        return wid * _BPW + i // _NCH, i % _NCH

    def load(i):
        b, c = item(i)
        k = i % 3
        pltpu.make_async_copy(
            x_hbm.at[b, pl.ds(c * _RC, _RC)], bufs[k], isems[k]).start()

    def load_wait(i):
        b, c = item(i)
        k = i % 3
        pltpu.make_async_copy(
            x_hbm.at[b, pl.ds(c * _RC, _RC)], bufs[k], isems[k]).wait()

    def store(i):
        b, c = item(i)
        k = i % 3
        pltpu.make_async_copy(
            bufs[k], out_hbm.at[b, pl.ds(c * _RC, _RC)], osems[k]).start()

    def store_wait(i):
        b, c = item(i)
        k = i % 3
        pltpu.make_async_copy(
            bufs[k], out_hbm.at[b, pl.ds(c * _RC, _RC)], osems[k]).wait()

    def prep_window(w):
        wload(w)
        wload_wait(w)
        fa, ta = wins[w]
        for (f, t) in holes[w // 2]:
            _patch_window(wbufs[0], fa, ta, f, t)

    load(0)
    load(1)
    prep_window(0)

    for i in range(n_items):
        load_wait(i)
        store(i)
        if i + 2 < n_items:
            if i - 1 >= 0:
                store_wait(i - 1)
                if i - 1 == _NCH - 1:
                    wstore(0)
                    wstore_wait(0)
                    prep_window(1)
                    wstore(1)
                    wstore_wait(1)
                    prep_window(2)
            load(i + 2)
    store_wait(n_items - 2)
    store_wait(n_items - 1)
    wstore(2)
    wstore_wait(2)
    prep_window(3)
    wstore(3)
    wstore_wait(3)


def kernel(x, f0, t0):
    meta = jnp.concatenate([
        f0.astype(jnp.int32),
        t0.astype(jnp.int32),
        jnp.zeros((_B, _ML - 4), jnp.int32),
    ], axis=1)
    mesh = plsc.VectorSubcoreMesh(core_axis_name="c", subcore_axis_name="s")
    fn = functools.partial(
        pl.kernel,
        out_type=jax.ShapeDtypeStruct((_B, _F, _T), jnp.float32),
        mesh=mesh,
        compiler_params=pltpu.CompilerParams(needs_layout_passes=False),
        scratch_types=(
            [pltpu.VMEM_SHARED((_NS, 3, _RC, _T), jnp.float32)]
            + [pltpu.VMEM((_WR, _WC), jnp.float32)]
            + [pltpu.VMEM((_B, _ML), jnp.int32)]
            + [pltpu.SemaphoreType.DMA for _ in range(8)]
        ),
    )(_sc_body)
    return fn(x, meta)
